# Initial kernel scaffold; baseline (speedup 1.0000x reference)
import jax
import jax.numpy as jnp
from jax.experimental import pallas as pl


def kernel(x, features, fps_key, W1, b1, g1, be1, W2, b2, g2, be2):
    raise NotImplementedError("write your pallas kernel here")


# speedup vs baseline: 5.9700x; 5.9700x over previous
import jax
import jax.numpy as jnp
from jax.experimental import pallas as pl
from jax.experimental.pallas import tpu as pltpu

_B, _N, _M, _K, _C, _D = 4, 4096, 1024, 32, 64, 3
_R2 = 0.15 ** 2
_OUT = 128
_HID = (_C + _D) * 4
_HID_PAD = 384
_IN_PAD = 128
_MB = 32
_KC = 8
_BIG = 1e9
_NS, _NL = 8, 512


def _fps_body(first_ref, xpl_ref, out_ref):
    xx = xpl_ref[0, 0]
    yy = xpl_ref[0, 1]
    zz = xpl_ref[0, 2]
    lin_n = (jax.lax.broadcasted_iota(jnp.int32, (_NS, _NL), 0) * _NL
             + jax.lax.broadcasted_iota(jnp.int32, (_NS, _NL), 1))
    lin_m = (jax.lax.broadcasted_iota(jnp.int32, (8, 128), 0) * 128
             + jax.lax.broadcasted_iota(jnp.int32, (8, 128), 1))
    b = pl.program_id(0)

    def extract(plane, far):
        return jnp.sum(jnp.where(lin_n == far, plane, 0.0))

    def dist_to(px, py, pz):
        dx = xx - px
        dy = yy - py
        dz = zz - pz
        return dx * dx + dy * dy + dz * dz

    far0 = first_ref[b]
    px = extract(xx, far0)
    py = extract(yy, far0)
    pz = extract(zz, far0)
    d = dist_to(px, py, pz)
    sx = jnp.where(lin_m == 0, px, 0.0)
    sy = jnp.where(lin_m == 0, py, 0.0)
    sz = jnp.where(lin_m == 0, pz, 0.0)

    def body(i, carry):
        d, sx, sy, sz = carry
        m = jnp.max(d)
        far = jnp.min(jnp.where(d == m, lin_n, _N))
        px = extract(xx, far)
        py = extract(yy, far)
        pz = extract(zz, far)
        nd = dist_to(px, py, pz)
        d = jnp.minimum(d, nd)
        sx = jnp.where(lin_m == i, px, sx)
        sy = jnp.where(lin_m == i, py, sy)
        sz = jnp.where(lin_m == i, pz, sz)
        return d, sx, sy, sz

    _, sx, sy, sz = jax.lax.fori_loop(1, _M, body, (d, sx, sy, sz))
    out_ref[0, 0] = sx
    out_ref[0, 1] = sy
    out_ref[0, 2] = sz


def _group_mlp_body(ctr_ref, xpl_ref, featx_ref, w1_ref, b1_ref, g1_ref,
                    be1_ref, w2_ref, b2_ref, g2_ref, be2_ref, out_ref):
    cx = ctr_ref[0, :, 0:1]
    cy = ctr_ref[0, :, 1:2]
    cz = ctr_ref[0, :, 2:3]
    xx = xpl_ref[0, 0:1, :]
    yy = xpl_ref[0, 1:2, :]
    zz = xpl_ref[0, 2:3, :]
    dx = cx - xx
    dy = cy - yy
    dz = cz - zz
    dist = dx * dx + dy * dy + dz * dz
    lin_n = jax.lax.broadcasted_iota(jnp.int32, (1, _N), 1)
    work = jnp.where(dist < _R2, dist, _BIG)

    idxs = []
    for _ in range(_K):
        m = jnp.min(work, axis=1, keepdims=True)
        sel = jnp.min(jnp.where(work == m, lin_n, _N), axis=1, keepdims=True)
        idxs.append(jnp.where(m < _BIG, sel, _N - 1))
        work = jnp.where(lin_n == sel, _BIG, work)

    lane = jax.lax.broadcasted_iota(jnp.int32, (1, _IN_PAD), 1)
    cpad = (jnp.where(lane == _C, cx, 0.0)
            + jnp.where(lane == _C + 1, cy, 0.0)
            + jnp.where(lane == _C + 2, cz, 0.0))

    featx = featx_ref[0]
    chunks = []
    for c0 in range(0, _K, _KC):
        onehot = jnp.concatenate(
            [(lin_n == idxs[k]).astype(jnp.float32) for k in range(c0, c0 + _KC)],
            axis=0)
        g = jnp.dot(onehot, featx, preferred_element_type=jnp.float32)
        g = (g.reshape(_KC, _MB, _IN_PAD) - cpad[None]).reshape(_KC * _MB, _IN_PAD)
        chunks.append(g)
    combined = jnp.concatenate(chunks, axis=0)

    h = jnp.dot(combined, w1_ref[...], preferred_element_type=jnp.float32) + b1_ref[...]
    h = jax.nn.gelu(h)
    hmask = (jax.lax.broadcasted_iota(jnp.int32, (1, _HID_PAD), 1) < _HID
             ).astype(jnp.float32)
    mu = jnp.sum(h, axis=1, keepdims=True) / _HID
    df = h - mu
    var = jnp.sum((df * hmask) ** 2, axis=1, keepdims=True) / _HID
    h = df / jnp.sqrt(var + 1e-6) * g1_ref[...] + be1_ref[...]

    h2 = jnp.dot(h, w2_ref[...], preferred_element_type=jnp.float32) + b2_ref[...]
    mu2 = jnp.sum(h2, axis=1, keepdims=True) / _OUT
    df2 = h2 - mu2
    var2 = jnp.sum(df2 * df2, axis=1, keepdims=True) / _OUT
    h2 = df2 / jnp.sqrt(var2 + 1e-6) * g2_ref[...] + be2_ref[...]

    out_ref[0] = jnp.max(h2.reshape(_K, _MB, _OUT), axis=0)


def kernel(x, features, fps_key, W1, b1, g1, be1, W2, b2, g2, be2):
    f32 = jnp.float32
    first_idx = jax.random.randint(fps_key, (_B,), 0, _N).astype(jnp.int32)
    xt = x.transpose(0, 2, 1)
    xpl = xt.reshape(_B, 3, _NS, _NL)

    sp_planes = pl.pallas_call(
        _fps_body,
        grid=(_B,),
        in_specs=[
            pl.BlockSpec(memory_space=pltpu.SMEM),
            pl.BlockSpec((1, 3, _NS, _NL), lambda b: (b, 0, 0, 0)),
        ],
        out_specs=pl.BlockSpec((1, 3, 8, 128), lambda b: (b, 0, 0, 0)),
        out_shape=jax.ShapeDtypeStruct((_B, 3, 8, 128), f32),
    )(first_idx, xpl)
    sampled_points = sp_planes.reshape(_B, 3, _M).transpose(0, 2, 1)

    featx = jnp.concatenate(
        [features, x, jnp.zeros((_B, _N, _IN_PAD - _C - _D), f32)], axis=-1)
    W1p = jnp.zeros((_IN_PAD, _HID_PAD), f32).at[:_C + _D, :_HID].set(W1)
    b1p = jnp.zeros((1, _HID_PAD), f32).at[0, :_HID].set(b1)
    g1p = jnp.zeros((1, _HID_PAD), f32).at[0, :_HID].set(g1)
    be1p = jnp.zeros((1, _HID_PAD), f32).at[0, :_HID].set(be1)
    W2p = jnp.zeros((_HID_PAD, _OUT), f32).at[:_HID, :].set(W2)

    wspec = lambda shp: pl.BlockSpec(shp, lambda b, m: (0,) * len(shp))
    out_features = pl.pallas_call(
        _group_mlp_body,
        grid=(_B, _M // _MB),
        in_specs=[
            pl.BlockSpec((1, _MB, 3), lambda b, m: (b, m, 0)),
            pl.BlockSpec((1, 3, _N), lambda b, m: (b, 0, 0)),
            pl.BlockSpec((1, _N, _IN_PAD), lambda b, m: (b, 0, 0)),
            wspec((_IN_PAD, _HID_PAD)),
            wspec((1, _HID_PAD)),
            wspec((1, _HID_PAD)),
            wspec((1, _HID_PAD)),
            wspec((_HID_PAD, _OUT)),
            wspec((1, _OUT)),
            wspec((1, _OUT)),
            wspec((1, _OUT)),
        ],
        out_specs=pl.BlockSpec((1, _MB, _OUT), lambda b, m: (b, m, 0)),
        out_shape=jax.ShapeDtypeStruct((_B, _M, _OUT), f32),
    )(sampled_points, xt, featx, W1p, b1p, g1p, be1p, W2p,
      b2.reshape(1, _OUT), g2.reshape(1, _OUT), be2.reshape(1, _OUT))

    return (sampled_points, out_features)
